# baseline (speedup 1.0000x reference)
import jax
import jax.numpy as jnp
from jax import lax
from jax.experimental import pallas as pl
from jax.experimental.pallas import tpu as pltpu

_B = 16384
_R = 128
_C = 128

_CONST_CACHE = {}


def _rd_constants():
    if "rd" not in _CONST_CACHE:
        rd = jax.random.normal(jax.random.key(42), (_B + 1,), jnp.float32)
        rd = jnp.maximum(jnp.sort(-jnp.exp(rd)), -1.0)
        rd0 = rd[:_B].reshape(_R, _C)
        rd1 = rd[1:].reshape(_R, _C)
        _CONST_CACHE["rd"] = (rd0, rd1)
    return _CONST_CACHE["rd"]


def _roll(x, shift, axis):
    return jnp.roll(x, shift, axis)


def _roll_stage(xs, bs, d, axis, dist, col8, sub8):
    it = col8 if axis == 1 else sub8
    is_hi = (it & dist) != 0
    is_lo = jnp.logical_not(is_hi)
    out = []
    if bs <= 512:
        for xi in xs:
            rm = _roll(xi, -dist, axis)
            rp = _roll(xi, dist, axis)
            out.append(jnp.where(is_lo, jnp.minimum(xi, rm),
                                 jnp.maximum(xi, rp)))
    else:
        for i, xi in enumerate(xs):
            rm = _roll(xi, -dist, axis)
            rp = _roll(xi, dist, axis)
            if (i & (bs // 1024)) != 0:
                out.append(jnp.where(is_lo, jnp.maximum(xi, rm),
                                     jnp.minimum(xi, rp)))
            else:
                out.append(jnp.where(is_lo, jnp.minimum(xi, rm),
                                     jnp.maximum(xi, rp)))
    return out


def _mxu_stage(xs, bs, d, col8, sub8, perms):
    xcat = jnp.concatenate(xs, axis=0)
    if d <= 64:
        p = jnp.dot(xcat, perms[d], preferred_element_type=jnp.float32)
        is_hi = (col8 & d) != 0
    else:
        dr = d // 128
        p = jnp.dot(perms[dr], xcat, preferred_element_type=jnp.float32)
        is_hi = (sub8 & dr) != 0
    is_lo = jnp.logical_not(is_hi)
    ps = [p[8 * i:8 * (i + 1), :] for i in range(16)]
    out = []
    if bs <= 512:
        for xi, pi in zip(xs, ps):
            out.append(jnp.where(is_lo, jnp.minimum(xi, pi),
                                 jnp.maximum(xi, pi)))
    else:
        for i, (xi, pi) in enumerate(zip(xs, ps)):
            if (i & (bs // 1024)) != 0:
                out.append(jnp.where(is_lo, jnp.maximum(xi, pi),
                                     jnp.minimum(xi, pi)))
            else:
                out.append(jnp.where(is_lo, jnp.minimum(xi, pi),
                                     jnp.maximum(xi, pi)))
    return out


def _vreg_stage(xs, bs, d):
    dv = d // 1024
    out = list(xs)
    for i in range(16):
        if (i & dv) == 0:
            j = i + dv
            mn = jnp.minimum(xs[i], xs[j])
            mx = jnp.maximum(xs[i], xs[j])
            if (i & (bs // 1024)) != 0:
                out[i], out[j] = mx, mn
            else:
                out[i], out[j] = mn, mx
    return out


def _bitonic_sort(v):
    xs = [v[8 * i:8 * (i + 1), :] for i in range(16)]
    sub8 = lax.broadcasted_iota(jnp.int32, (8, _C), 0)
    col8 = lax.broadcasted_iota(jnp.int32, (8, _C), 1)

    def _sign8(bs):
        bit = (col8 & bs) if bs <= 64 else (sub8 & (bs // 128))
        return jnp.where(bit != 0, -1.0, 1.0).astype(jnp.float32)

    row_f = lax.broadcasted_iota(jnp.int32, (_R, _C), 0)
    col_f = lax.broadcasted_iota(jnp.int32, (_R, _C), 1)
    perms = {m: ((row_f ^ m) == col_f).astype(jnp.float32)
             for m in (1, 2, 4, 8, 16, 32, 64)}

    for k in range(1, 15):
        bs = 1 << k
        if bs <= 512:
            sg = _sign8(bs)
            xs = [xi * sg for xi in xs]
        for j in range(k - 1, -1, -1):
            d = 1 << j
            if d <= 512:
                xs = _mxu_stage(xs, bs, d, col8, sub8, perms)
            else:
                xs = _vreg_stage(xs, bs, d)
        if bs <= 512:
            xs = [xi * sg for xi in xs]
    return jnp.concatenate(xs, axis=0)


def _body(out_ref, tgt_ref, rd0_ref, rd1_ref, loss_ref):
    outp = out_ref[...]
    tgt = tgt_ref[...].astype(jnp.float32)
    s = outp * (2.0 * tgt - 1.0)
    v = jnp.minimum(s, 1.0)
    s_sum = jnp.sum(s)
    r0 = jnp.sum((s < 0.0).astype(jnp.int32))
    w = _bitonic_sort(v)
    row = lax.broadcasted_iota(jnp.int32, (_R, _C), 0)
    col = lax.broadcasted_iota(jnp.int32, (_R, _C), 1)
    k = row * _C + col
    sel = jnp.where(k < r0, rd0_ref[...], rd1_ref[...])
    loss = (jnp.sum(sel * w) - 1e-6 * s_sum) / _B
    loss_ref[0, 0] = loss


def _pallas_loss(outp, tgt, rd0, rd1, interpret=False):
    return pl.pallas_call(
        _body,
        out_shape=jax.ShapeDtypeStruct((1, 1), jnp.float32),
        out_specs=pl.BlockSpec(memory_space=pltpu.SMEM),
        interpret=interpret,
    )(outp, tgt, rd0, rd1)


def kernel(output, target, interpret=False):
    rd0, rd1 = _rd_constants()
    outp = output.reshape(_R, _C).astype(jnp.float32)
    tgt = target.reshape(_R, _C).astype(jnp.int32)
    res = _pallas_loss(outp, tgt, rd0, rd1, interpret=interpret)
    return res[0, 0]


# speedup vs baseline: 1.0775x; 1.0775x over previous
import jax
import jax.numpy as jnp
from jax import lax
from jax.experimental import pallas as pl
from jax.experimental.pallas import tpu as pltpu

_B = 16384
_R = 128
_C = 128

_CONST_CACHE = {}


def _rd_constants():
    if "rd" not in _CONST_CACHE:
        rd = jax.random.normal(jax.random.key(42), (_B + 1,), jnp.float32)
        rd = jnp.maximum(jnp.sort(-jnp.exp(rd)), -1.0)
        rd0 = rd[:_B].reshape(_R, _C)
        rd1 = rd[1:].reshape(_R, _C)
        _CONST_CACHE["rd"] = (rd0, rd1)
    return _CONST_CACHE["rd"]


def _roll(x, shift, axis):
    return jnp.roll(x, shift, axis)


def _roll_stage(xs, bs, d, axis, dist, col8, sub8):
    it = col8 if axis == 1 else sub8
    is_hi = (it & dist) != 0
    is_lo = jnp.logical_not(is_hi)
    out = []
    if bs <= 512:
        for xi in xs:
            rm = _roll(xi, -dist, axis)
            rp = _roll(xi, dist, axis)
            out.append(jnp.where(is_lo, jnp.minimum(xi, rm),
                                 jnp.maximum(xi, rp)))
    else:
        for i, xi in enumerate(xs):
            rm = _roll(xi, -dist, axis)
            rp = _roll(xi, dist, axis)
            if (i & (bs // 1024)) != 0:
                out.append(jnp.where(is_lo, jnp.maximum(xi, rm),
                                     jnp.minimum(xi, rp)))
            else:
                out.append(jnp.where(is_lo, jnp.minimum(xi, rm),
                                     jnp.maximum(xi, rp)))
    return out


def _mxu_stage(xs, bs, d, col8, sub8, perms):
    xcat = jnp.concatenate(xs, axis=0)
    if d <= 64:
        p = jnp.dot(xcat, perms[d], preferred_element_type=jnp.float32)
        is_hi = (col8 & d) != 0
    else:
        dr = d // 128
        p = jnp.dot(perms[dr], xcat, preferred_element_type=jnp.float32)
        is_hi = (sub8 & dr) != 0
    is_lo = jnp.logical_not(is_hi)
    ps = [p[8 * i:8 * (i + 1), :] for i in range(16)]
    out = []
    if bs <= 512:
        for xi, pi in zip(xs, ps):
            out.append(jnp.where(is_lo, jnp.minimum(xi, pi),
                                 jnp.maximum(xi, pi)))
    else:
        for i, (xi, pi) in enumerate(zip(xs, ps)):
            if (i & (bs // 1024)) != 0:
                out.append(jnp.where(is_lo, jnp.maximum(xi, pi),
                                     jnp.minimum(xi, pi)))
            else:
                out.append(jnp.where(is_lo, jnp.minimum(xi, pi),
                                     jnp.maximum(xi, pi)))
    return out


def _lane_hybrid(xs, bs, d, col8, col_iota_perms):
    is_hi = (col8 & d) != 0
    is_lo = jnp.logical_not(is_hi)
    xcat = jnp.concatenate(xs[8:], axis=0)
    p = jnp.dot(xcat, col_iota_perms[d], preferred_element_type=jnp.float32)
    out = []
    for i, xi in enumerate(xs):
        flip = bs > 512 and (i & (bs // 1024)) != 0
        if i < 8:
            p_lo, p_hi = _roll(xi, -d, 1), _roll(xi, d, 1)
        else:
            p_lo = p_hi = p[8 * (i - 8):8 * (i - 7), :]
        if flip:
            out.append(jnp.where(is_lo, jnp.maximum(xi, p_lo),
                                 jnp.minimum(xi, p_hi)))
        else:
            out.append(jnp.where(is_lo, jnp.minimum(xi, p_lo),
                                 jnp.maximum(xi, p_hi)))
    return out


def _vreg_stage(xs, bs, d):
    dv = d // 1024
    out = list(xs)
    for i in range(16):
        if (i & dv) == 0:
            j = i + dv
            mn = jnp.minimum(xs[i], xs[j])
            mx = jnp.maximum(xs[i], xs[j])
            if (i & (bs // 1024)) != 0:
                out[i], out[j] = mx, mn
            else:
                out[i], out[j] = mn, mx
    return out


def _bitonic_sort(v):
    xs = [v[8 * i:8 * (i + 1), :] for i in range(16)]
    sub8 = lax.broadcasted_iota(jnp.int32, (8, _C), 0)
    col8 = lax.broadcasted_iota(jnp.int32, (8, _C), 1)

    def _sign8(bs):
        bit = (col8 & bs) if bs <= 64 else (sub8 & (bs // 128))
        return jnp.where(bit != 0, -1.0, 1.0).astype(jnp.float32)

    row_f = lax.broadcasted_iota(jnp.int32, (_R, _C), 0)
    col_f = lax.broadcasted_iota(jnp.int32, (_R, _C), 1)
    perms = {m: ((row_f ^ m) == col_f).astype(jnp.float32)
             for m in (1, 2, 4, 8, 16, 32, 64)}

    for k in range(1, 15):
        bs = 1 << k
        if bs <= 512:
            sg = _sign8(bs)
            xs = [xi * sg for xi in xs]
        for j in range(k - 1, -1, -1):
            d = 1 << j
            if d <= 64:
                xs = _lane_hybrid(xs, bs, d, col8, perms)
            elif d <= 512:
                xs = _roll_stage(xs, bs, d, 0, d // 128, col8, sub8)
            else:
                xs = _vreg_stage(xs, bs, d)
        if bs <= 512:
            xs = [xi * sg for xi in xs]
    return jnp.concatenate(xs, axis=0)


def _body(out_ref, tgt_ref, rd0_ref, rd1_ref, loss_ref):
    outp = out_ref[...]
    tgt = tgt_ref[...].astype(jnp.float32)
    s = outp * (2.0 * tgt - 1.0)
    v = jnp.minimum(s, 1.0)
    s_sum = jnp.sum(s)
    r0 = jnp.sum((s < 0.0).astype(jnp.int32))
    w = _bitonic_sort(v)
    row = lax.broadcasted_iota(jnp.int32, (_R, _C), 0)
    col = lax.broadcasted_iota(jnp.int32, (_R, _C), 1)
    k = row * _C + col
    sel = jnp.where(k < r0, rd0_ref[...], rd1_ref[...])
    loss = (jnp.sum(sel * w) - 1e-6 * s_sum) / _B
    loss_ref[0, 0] = loss


def _pallas_loss(outp, tgt, rd0, rd1, interpret=False):
    return pl.pallas_call(
        _body,
        out_shape=jax.ShapeDtypeStruct((1, 1), jnp.float32),
        out_specs=pl.BlockSpec(memory_space=pltpu.SMEM),
        interpret=interpret,
    )(outp, tgt, rd0, rd1)


def kernel(output, target, interpret=False):
    rd0, rd1 = _rd_constants()
    outp = output.reshape(_R, _C).astype(jnp.float32)
    tgt = target.reshape(_R, _C).astype(jnp.int32)
    res = _pallas_loss(outp, tgt, rd0, rd1, interpret=interpret)
    return res[0, 0]


# speedup vs baseline: 1.2222x; 1.1343x over previous
import jax
import jax.numpy as jnp
from jax import lax
from jax.experimental import pallas as pl
from jax.experimental.pallas import tpu as pltpu

_B = 16384
_R = 128
_C = 128

_CONST_CACHE = {}


def _rd_constants():
    if "rd" not in _CONST_CACHE:
        rd = jax.random.normal(jax.random.key(42), (_B + 1,), jnp.float32)
        rd = jnp.maximum(jnp.sort(-jnp.exp(rd)), -1.0)
        rd0 = rd[:_B].reshape(_R, _C)
        rd1 = rd[1:].reshape(_R, _C)
        _CONST_CACHE["rd"] = (rd0, rd1)
    return _CONST_CACHE["rd"]


def _roll(x, shift, axis):
    return jnp.roll(x, shift, axis)


def _roll_stage(xs, bs, d, axis, dist, col8, sub8):
    it = col8 if axis == 1 else sub8
    is_hi = (it & dist) != 0
    is_lo = jnp.logical_not(is_hi)
    out = []
    if bs <= 512:
        for xi in xs:
            rm = _roll(xi, -dist, axis)
            rp = _roll(xi, dist, axis)
            out.append(jnp.where(is_lo, jnp.minimum(xi, rm),
                                 jnp.maximum(xi, rp)))
    else:
        for i, xi in enumerate(xs):
            rm = _roll(xi, -dist, axis)
            rp = _roll(xi, dist, axis)
            if (i & (bs // 1024)) != 0:
                out.append(jnp.where(is_lo, jnp.maximum(xi, rm),
                                     jnp.minimum(xi, rp)))
            else:
                out.append(jnp.where(is_lo, jnp.minimum(xi, rm),
                                     jnp.maximum(xi, rp)))
    return out


def _vreg_stage(xs, bs, d):
    dv = d // 1024
    out = list(xs)
    for i in range(16):
        if (i & dv) == 0:
            j = i + dv
            mn = jnp.minimum(xs[i], xs[j])
            mx = jnp.maximum(xs[i], xs[j])
            if (i & (bs // 1024)) != 0:
                out[i], out[j] = mx, mn
            else:
                out[i], out[j] = mn, mx
    return out


def _bitonic_sort(v):
    xs = [v[8 * i:8 * (i + 1), :] for i in range(16)]
    sub8 = lax.broadcasted_iota(jnp.int32, (8, _C), 0)
    col8 = lax.broadcasted_iota(jnp.int32, (8, _C), 1)

    def _sign8(bs):
        bit = (col8 & bs) if bs <= 64 else (sub8 & (bs // 128))
        return jnp.where(bit != 0, -1.0, 1.0).astype(jnp.float32)

    for k in range(1, 15):
        bs = 1 << k
        if bs <= 512:
            sg = _sign8(bs)
            xs = [xi * sg for xi in xs]
        for j in range(k - 1, -1, -1):
            d = 1 << j
            if d <= 64:
                xs = _roll_stage(xs, bs, d, 1, d, col8, sub8)
            elif d <= 512:
                xs = _roll_stage(xs, bs, d, 0, d // 128, col8, sub8)
            else:
                xs = _vreg_stage(xs, bs, d)
        if bs <= 512:
            xs = [xi * sg for xi in xs]
    return jnp.concatenate(xs, axis=0)


def _body(out_ref, tgt_ref, rd0_ref, rd1_ref, loss_ref):
    outp = out_ref[...]
    tgt = tgt_ref[...].astype(jnp.float32)
    s = outp * (2.0 * tgt - 1.0)
    v = jnp.minimum(s, 1.0)
    s_sum = jnp.sum(s)
    r0 = jnp.sum((s < 0.0).astype(jnp.int32))
    w = _bitonic_sort(v)
    row = lax.broadcasted_iota(jnp.int32, (_R, _C), 0)
    col = lax.broadcasted_iota(jnp.int32, (_R, _C), 1)
    k = row * _C + col
    sel = jnp.where(k < r0, rd0_ref[...], rd1_ref[...])
    loss = (jnp.sum(sel * w) - 1e-6 * s_sum) / _B
    loss_ref[0, 0] = loss


def _pallas_loss(outp, tgt, rd0, rd1, interpret=False):
    return pl.pallas_call(
        _body,
        out_shape=jax.ShapeDtypeStruct((1, 1), jnp.float32),
        out_specs=pl.BlockSpec(memory_space=pltpu.SMEM),
        interpret=interpret,
    )(outp, tgt, rd0, rd1)


def kernel(output, target, interpret=False):
    rd0, rd1 = _rd_constants()
    outp = output.reshape(_R, _C).astype(jnp.float32)
    tgt = target.reshape(_R, _C).astype(jnp.int32)
    res = _pallas_loss(outp, tgt, rd0, rd1, interpret=interpret)
    return res[0, 0]
